# baseline (speedup 1.0000x reference)
import jax
import jax.numpy as jnp
from jax import lax
from jax.experimental import pallas as pl
from jax.experimental.pallas import tpu as pltpu
from jax.experimental.pallas import tpu_sc as plsc

N_NODES = 10000
N_EDGES = 320000
H = 128
NPAD = 10240
EPAD = 327680
PW = 16
NC, NS = 2, 16
NW = NC * NS
CH = 128
ET = 512
NT = 512


def _mesh():
    return plsc.VectorSubcoreMesh(
        core_axis_name="c", subcore_axis_name="s", num_cores=NC, num_subcores=NS
    )


def _sc_gather(tables, idxs):
    n = len(tables)
    e = idxs[0].shape[0]
    epw = e // NW
    nch = epw // CH
    w = int(tables[0].shape[1])
    out_type = tuple(jax.ShapeDtypeStruct((e, w), jnp.float32)
                     for _ in range(n))
    scratch = []
    for _ in range(n):
        scratch.append(pltpu.VMEM((CH,), jnp.int32))
        scratch.append(pltpu.VMEM((CH, w), jnp.float32))
    scratch.append(pltpu.SemaphoreType.DMA)

    def body(*refs):
        table_refs = refs[:n]
        idx_refs = refs[n:2 * n]
        out_refs = refs[2 * n:3 * n]
        sc = refs[3 * n:]
        sem = sc[-1]
        wid = lax.axis_index("s") * NC + lax.axis_index("c")
        base = wid * epw

        def chunk(k, carry):
            off = base + k * CH
            for s in range(n):
                iv, rv = sc[2 * s], sc[2 * s + 1]
                pltpu.sync_copy(idx_refs[s].at[pl.ds(off, CH)], iv)
                pltpu.async_copy(table_refs[s].at[iv], rv, sem).wait()
                pltpu.sync_copy(rv, out_refs[s].at[pl.ds(off, CH)])
            return carry

        lax.fori_loop(0, nch, chunk, 0)

    fn = pl.kernel(body, out_type=out_type, mesh=_mesh(), scratch_types=scratch)
    return fn(*tables, *idxs)


def _sc_scatter(vals, idx, zeros):
    e = idx.shape[0]
    w = int(vals.shape[1])
    epw = e // NW
    nch = epw // CH
    rpt = NPAD // NS
    out_type = jax.ShapeDtypeStruct((NC * NPAD, w), jnp.float32)
    scratch = [
        pltpu.VMEM((CH,), jnp.int32),
        pltpu.VMEM((CH, w), jnp.float32),
        pltpu.VMEM_SHARED((NPAD, w), jnp.float32),
    ]

    def body(val_hbm, idx_hbm, zeros_hbm, out_hbm, iv, vv, shared):
        cid = lax.axis_index("c")
        sid = lax.axis_index("s")
        wid = sid * NC + cid
        base = wid * epw
        rs = sid * rpt
        pltpu.sync_copy(zeros_hbm.at[pl.ds(rs, rpt)], shared.at[pl.ds(rs, rpt)])
        plsc.subcore_barrier()

        def chunk(k, carry):
            off = base + k * CH
            pltpu.sync_copy(idx_hbm.at[pl.ds(off, CH)], iv)
            pltpu.sync_copy(val_hbm.at[pl.ds(off, CH)], vv)
            pltpu.sync_copy(vv, shared.at[iv], add=True)
            return carry

        lax.fori_loop(0, nch, chunk, 0)
        plsc.subcore_barrier()
        pltpu.sync_copy(shared.at[pl.ds(rs, rpt)],
                        out_hbm.at[pl.ds(cid * NPAD + rs, rpt)])

    fn = pl.kernel(body, out_type=out_type, mesh=_mesh(), scratch_types=scratch)
    return fn(vals, idx, zeros)


def _full(shape):
    return pl.BlockSpec(shape, lambda i: (0,) * len(shape))


def _tc_lengths(pr, pc):
    e = pr.shape[0]

    def f(pr_ref, pc_ref, out):
        d = pr_ref[...] - pc_ref[...]
        d0, d1, d2 = d[:, 0:1], d[:, 1:2], d[:, 2:3]
        ssq = (d0 * d0 + d2 * d2) + d1 * d1
        el = jnp.sqrt(jnp.maximum(ssq, 1e-30))
        out[...] = jnp.broadcast_to(el, (d.shape[0], PW))

    return pl.pallas_call(
        f, grid=(e // ET,),
        in_specs=[pl.BlockSpec((ET, H), lambda i: (i, 0))] * 2,
        out_specs=pl.BlockSpec((ET, PW), lambda i: (i, 0)),
        out_shape=jax.ShapeDtypeStruct((e, PW), jnp.float32),
    )(pr, pc)


def _tc_edge_enc(el16, et16, w1, b1, w2, b2, bond):
    e = el16.shape[0]

    def f(el_ref, et_ref, w1r, b1r, w2r, b2r, br, out):
        el = el_ref[:, :1]
        t = jnp.maximum(el * w1r[...] + b1r[...], 0.0)
        d = jnp.dot(t, w2r[...], preferred_element_type=jnp.float32) + b2r[...]
        oh = (et_ref[:, :1] ==
              lax.broadcasted_iota(jnp.int32, (el.shape[0], 128), 1)
              ).astype(jnp.float32)
        emb = jnp.dot(oh, br[...], preferred_element_type=jnp.float32,
                      precision=lax.Precision.HIGHEST)
        out[...] = d * emb

    return pl.pallas_call(
        f, grid=(e // ET,),
        in_specs=[
            pl.BlockSpec((ET, PW), lambda i: (i, 0)),
            pl.BlockSpec((ET, PW), lambda i: (i, 0)),
            _full((1, H)), _full((1, H)), _full((H, H)), _full((1, H)),
            _full((128, H)),
        ],
        out_specs=pl.BlockSpec((ET, H), lambda i: (i, 0)),
        out_shape=jax.ShapeDtypeStruct((e, H), jnp.float32),
    )(el16, et16, w1, b1, w2, b2, bond)


def _tc_embed(x, w, b):
    n = x.shape[0]

    def f(xr, wr, br, out):
        out[...] = jnp.dot(xr[...], wr[...],
                           preferred_element_type=jnp.float32) + br[...]

    return pl.pallas_call(
        f, grid=(n // NT,),
        in_specs=[pl.BlockSpec((NT, H), lambda i: (i, 0)),
                  _full((H, H)), _full((1, H))],
        out_specs=pl.BlockSpec((NT, H), lambda i: (i, 0)),
        out_shape=jax.ShapeDtypeStruct((n, H), jnp.float32),
    )(x, w, b)


def _tc_msg(hr, hc, ea, w1, b1, w2, b2):
    e = hr.shape[0]

    def f(hr_ref, hc_ref, ea_ref, w1r, b1r, w2r, b2r, out):
        t = (jnp.dot(hr_ref[...], w1r[:H], preferred_element_type=jnp.float32)
             + jnp.dot(hc_ref[...], w1r[H:2 * H],
                       preferred_element_type=jnp.float32))
        t = t + jnp.dot(ea_ref[...], w1r[2 * H:],
                        preferred_element_type=jnp.float32)
        m = jnp.maximum(t + b1r[...], 0.0)
        m = jnp.maximum(
            jnp.dot(m, w2r[...], preferred_element_type=jnp.float32)
            + b2r[...], 0.0)
        out[...] = m

    return pl.pallas_call(
        f, grid=(e // ET,),
        in_specs=[pl.BlockSpec((ET, H), lambda i: (i, 0))] * 3 + [
            _full((3 * H, H)), _full((1, H)), _full((H, H)), _full((1, H))],
        out_specs=pl.BlockSpec((ET, H), lambda i: (i, 0)),
        out_shape=jax.ShapeDtypeStruct((e, H), jnp.float32),
    )(hr, hc, ea, w1, b1, w2, b2)


def _tc_node(h, acc, w1, b1, w2, b2):
    n = h.shape[0]

    def f(h_ref, a_ref, w1r, b1r, w2r, b2r, out):
        hv = h_ref[...]
        agg = a_ref[0] + a_ref[1]
        t = (jnp.dot(hv, w1r[:H], preferred_element_type=jnp.float32)
             + jnp.dot(agg, w1r[H:], preferred_element_type=jnp.float32))
        t = jnp.maximum(t + b1r[...], 0.0)
        t = jnp.dot(t, w2r[...], preferred_element_type=jnp.float32) + b2r[...]
        out[...] = hv + t

    acc3 = acc.reshape(NC, n, H)
    return pl.pallas_call(
        f, grid=(n // NT,),
        in_specs=[pl.BlockSpec((NT, H), lambda i: (i, 0)),
                  pl.BlockSpec((NC, NT, H), lambda i: (0, i, 0)),
                  _full((2 * H, H)), _full((1, H)), _full((H, H)),
                  _full((1, H))],
        out_specs=pl.BlockSpec((NT, H), lambda i: (i, 0)),
        out_shape=jax.ShapeDtypeStruct((n, H), jnp.float32),
    )(h, acc3, w1, b1, w2, b2)


def _tc_grad(nar, nac, ea, pr, pc, el16, w1, b1, w2, b2, w3r, b3):
    e = nar.shape[0]

    def f(nar_ref, nac_ref, ea_ref, pr_ref, pc_ref, el_ref,
          w1r, b1r, w2r, b2r, w3rr, b3r, out):
        t = (jnp.dot(nar_ref[...], w1r[:H], preferred_element_type=jnp.float32)
             + jnp.dot(nac_ref[...], w1r[H:2 * H],
                       preferred_element_type=jnp.float32))
        t = t + jnp.dot(ea_ref[...], w1r[2 * H:],
                        preferred_element_type=jnp.float32)
        x = jnp.maximum(t + b1r[...], 0.0)
        x = jnp.maximum(
            jnp.dot(x, w2r[...], preferred_element_type=jnp.float32)
            + b2r[...], 0.0)
        einv = jnp.dot(x, w3rr[...],
                       preferred_element_type=jnp.float32)[:, :1] + b3r[:, :1]
        dd = (1.0 / el_ref[:, :1]) * (pr_ref[...] - pc_ref[...])
        out[...] = dd * einv

    return pl.pallas_call(
        f, grid=(e // ET,),
        in_specs=[pl.BlockSpec((ET, H), lambda i: (i, 0))] * 5 + [
            pl.BlockSpec((ET, PW), lambda i: (i, 0)),
            _full((3 * H, H)), _full((1, H)), _full((H, H // 2)),
            _full((1, H // 2)), _full((H // 2, H)), _full((1, H // 2))],
        out_specs=pl.BlockSpec((ET, H), lambda i: (i, 0)),
        out_shape=jax.ShapeDtypeStruct((e, H), jnp.float32),
    )(nar, nac, ea, pr, pc, el16, w1, b1, w2, b2, w3r, b3)


def _tc_posup(posp, acc):
    n = posp.shape[0]

    def f(p_ref, a_ref, out):
        out[...] = p_ref[...] + (a_ref[0] + a_ref[1]) / 100.0

    acc3 = acc.reshape(NC, n, H)
    return pl.pallas_call(
        f, grid=(n // NT,),
        in_specs=[pl.BlockSpec((NT, H), lambda i: (i, 0)),
                  pl.BlockSpec((NC, NT, H), lambda i: (0, i, 0))],
        out_specs=pl.BlockSpec((NT, H), lambda i: (i, 0)),
        out_shape=jax.ShapeDtypeStruct((n, H), jnp.float32),
    )(posp, acc3)


def kernel(node_emb, node_type, node_degree, pos, edge_index, edge_type,
           batch, time_step, params):
    ne = edge_index.shape[1]
    npe = EPAD - ne
    dump = jnp.full((npe,), NPAD - 1, jnp.int32)
    row = jnp.concatenate([edge_index[0].astype(jnp.int32), dump])
    col = jnp.concatenate([edge_index[1].astype(jnp.int32), dump])
    et16 = jnp.broadcast_to(
        jnp.concatenate([edge_type.astype(jnp.int32),
                         jnp.zeros((npe,), jnp.int32)])[:, None], (EPAD, PW))
    posp = jnp.zeros((NPAD, H), jnp.float32).at[:N_NODES, :3].set(pos)
    xpad = jnp.zeros((NPAD, H), jnp.float32).at[:N_NODES].set(node_emb)
    zerosH = jnp.zeros((NPAD, H), jnp.float32)

    pr0, pc0 = _sc_gather((posp, posp), (row, col))
    el16 = _tc_lengths(pr0, pc0)

    for bp in params['blocks']:
        ee, g, gm = bp['edge_enc'], bp['gcl'], bp['grad_mlp']
        nbond = ee['bond_emb'].shape[0]
        bond = jnp.zeros((128, H), jnp.float32).at[:nbond].set(ee['bond_emb'])
        ea = _tc_edge_enc(el16, et16, ee['mW1'], ee['mb1'][None],
                          ee['mW2'], ee['mb2'][None], bond)
        h = _tc_embed(xpad, g['embed_W'], g['embed_b'][None])
        for cv in g['convs']:
            hr, hc = _sc_gather((h, h), (row, col))
            m2 = _tc_msg(hr, hc, ea, cv['eW1'], cv['eb1'][None],
                         cv['eW2'], cv['eb2'][None])
            acc = _sc_scatter(m2, row, zerosH)
            h = _tc_node(h, acc, cv['nW1'], cv['nb1'][None],
                         cv['nW2'], cv['nb2'][None])
        nar, nac, prb, pcb = _sc_gather((h, h, posp, posp),
                                        (row, col, row, col))
        b3 = jnp.broadcast_to(gm['gb3'][None], (1, H // 2))
        w3p = jnp.zeros((H // 2, H), jnp.float32).at[:, :1].set(gm['gW3'])
        tr = _tc_grad(nar, nac, ea, prb, pcb, el16,
                      gm['gW1'], gm['gb1'][None], gm['gW2'], gm['gb2'][None],
                      w3p, b3)
        acct = _sc_scatter(tr, row, zerosH)
        posp = _tc_posup(posp, acct)

    return posp[:N_NODES, :3] - pos
